# Initial kernel scaffold; baseline (speedup 1.0000x reference)
import jax
import jax.numpy as jnp
from jax.experimental import pallas as pl


def kernel(arguments, DKG, drugEmb, relEmb, tailEmb, W1, b1, W2, b2, gamma, beta, W3, W4, W5):
    raise NotImplementedError("write your pallas kernel here")


# speedup vs baseline: 4.2177x; 4.2177x over previous
import functools

import jax
import jax.numpy as jnp
from jax import lax
from jax.experimental import pallas as pl
from jax.experimental.pallas import tpu as pltpu
from jax.experimental.pallas import tpu_sc as plsc

N = 10000
NP = 10240
E = 320000
D = 128
W = 144
NC = 2
NS = 16
EPW = E // (NC * NS)
C = 80
NCHUNK = EPW // C
RPT = NP // NS


def _prep_body(drugEmb, tailEmb, relEmb, W1, b1, W3, W4, W5,
               drug2, tail2, rel2):
    w1s = jnp.sum(W1[...], axis=1)
    rel2[...] = relEmb[...] * w1s[None, :]
    w5 = W5[...]
    w35 = W3[...] @ w5[:D]
    w45 = W4[...] @ w5[D:]
    p = drugEmb[...] @ w35
    q = tailEmb[...] @ w45
    b1s = jnp.sum(b1[...])
    col = lax.broadcasted_iota(jnp.int32, (N, W - D), 1)
    dext = jnp.where(col == 0, p, 0.0) + jnp.where(col == 1, b1s, 0.0)
    text = jnp.where(col == 0, q, 0.0)
    drug2[:, :D] = drugEmb[...]
    drug2[:, D:] = dext
    tail2[:, :D] = tailEmb[...]
    tail2[:, D:] = text


def _prep(drugEmb, tailEmb, relEmb, W1, b1, W3, W4, W5):
    return pl.pallas_call(
        _prep_body,
        out_shape=(
            jax.ShapeDtypeStruct((N, W), jnp.float32),
            jax.ShapeDtypeStruct((N, W), jnp.float32),
            jax.ShapeDtypeStruct((N, D), jnp.float32),
        ),
    )(drugEmb, tailEmb, relEmb, W1, b1, W3, W4, W5)


def _edge_body(drug2, tail2, rel2, heads, tails, rels, out,
               acc, dbuf, tbuf, rbuf, hidx, tidx, ridx,
               sem0, sem1, sem2):
    c = lax.axis_index("c")
    s = lax.axis_index("s")

    lanes = lax.iota(jnp.int32, 16)
    z16 = jnp.zeros((16,), jnp.float32)
    c128 = jnp.full((16,), D, jnp.int32)
    c129 = jnp.full((16,), D + 1, jnp.int32)

    def _zrow(i, carry):
        def _zcol(k, carry2):
            plsc.store_scatter(tbuf, [i * 16 + lanes, jnp.broadcast_to(k, (16,))], z16)
            return carry2
        return lax.fori_loop(0, W, _zcol, carry)
    lax.fori_loop(0, C // 16, _zrow, 0)

    base = s * RPT
    for j in range(RPT // C):
        pltpu.sync_copy(tbuf.at[pl.ds(0, C)], acc.at[pl.ds(base + j * C, C)])
    plsc.subcore_barrier()

    ebase = (c * NS + s) * EPW

    def _chunk(ci, carry):
        off = ebase + ci * C
        pltpu.sync_copy(heads.at[pl.ds(off, C)], hidx)
        pltpu.sync_copy(tails.at[pl.ds(off, C)], tidx)
        pltpu.sync_copy(rels.at[pl.ds(off, C)], ridx)
        cp0 = pltpu.async_copy(drug2.at[hidx], dbuf, sem0)
        cp1 = pltpu.async_copy(tail2.at[tidx], tbuf, sem1)
        cp2 = pltpu.async_copy(rel2.at[ridx], rbuf, sem2)
        cp0.wait()
        cp1.wait()
        cp2.wait()

        def _group(g, carry2):
            rows = g * 16 + lanes
            ph = plsc.load_gather(dbuf, [rows, c128])
            qt = plsc.load_gather(tbuf, [rows, c128])
            x = ph + qt
            att = jnp.maximum(x, 0.01 * x)
            eatt = jnp.exp(att)

            def _dot(d, a):
                dd = jnp.broadcast_to(d, (16,))
                return a + (plsc.load_gather(dbuf, [rows, dd]) *
                            plsc.load_gather(rbuf, [rows, dd]))
            dotA = lax.fori_loop(0, D, _dot, jnp.zeros((16,), jnp.float32),
                                 unroll=16)
            b1v = plsc.load_gather(dbuf, [rows, c129])
            wv = eatt * (dotA + b1v)

            def _out(d, carry3):
                dd = jnp.broadcast_to(d, (16,))
                tv = plsc.load_gather(tbuf, [rows, dd])
                plsc.store_scatter(tbuf, [rows, dd], wv * tv)
                return carry3
            lax.fori_loop(0, D, _out, 0, unroll=16)
            plsc.store_scatter(tbuf, [rows, c128], eatt)
            return carry2
        lax.fori_loop(0, C // 16, _group, 0)

        pltpu.sync_copy(tbuf, acc.at[hidx], add=True)
        return carry
    lax.fori_loop(0, NCHUNK, _chunk, 0)

    plsc.subcore_barrier()
    pltpu.sync_copy(acc.at[pl.ds(base, RPT)], out.at[c, pl.ds(base, RPT)])


_edge_kernel = functools.partial(
    pl.kernel,
    out_type=jax.ShapeDtypeStruct((NC, NP, W), jnp.float32),
    mesh=plsc.VectorSubcoreMesh(core_axis_name="c", subcore_axis_name="s",
                                num_cores=NC, num_subcores=NS),
    compiler_params=pltpu.CompilerParams(use_tc_tiling_on_sc=False,
                                         needs_layout_passes=False),
    scratch_types=[
        pltpu.VMEM_SHARED((NP, W), jnp.float32),
        pltpu.VMEM((C, W), jnp.float32),
        pltpu.VMEM((C, W), jnp.float32),
        pltpu.VMEM((C, D), jnp.float32),
        pltpu.VMEM((C,), jnp.int32),
        pltpu.VMEM((C,), jnp.int32),
        pltpu.VMEM((C,), jnp.int32),
        pltpu.SemaphoreType.DMA,
        pltpu.SemaphoreType.DMA,
        pltpu.SemaphoreType.DMA,
    ],
)(_edge_body)


def _finish_body(part, drugEmb, W2, b2, gamma, beta, out):
    acc = part[0, :N] + part[1, :N]
    denom = acc[:, D:D + 1]
    neigh = acc[:, :D] / jnp.where(denom == 0.0, 1.0, denom)
    lin = drugEmb[...] @ W2[:D] + neigh @ W2[D:] + b2[...][None, :]
    mu = jnp.mean(lin, axis=0, keepdims=True)
    var = jnp.mean((lin - mu) ** 2, axis=0, keepdims=True)
    out[...] = (gamma[...][None, :] * (lin - mu) / jnp.sqrt(var + 1e-5)
                + beta[...][None, :])


def _finish(part, drugEmb, W2, b2, gamma, beta):
    return pl.pallas_call(
        _finish_body,
        out_shape=jax.ShapeDtypeStruct((N, D), jnp.float32),
    )(part, drugEmb, W2, b2, gamma, beta)


def kernel(arguments, DKG, drugEmb, relEmb, tailEmb, W1, b1, W2, b2,
           gamma, beta, W3, W4, W5):
    heads = DKG[:, 0]
    tails = DKG[:, 1]
    rels = DKG[:, 2]
    drug2, tail2, rel2 = _prep(drugEmb, tailEmb, relEmb, W1, b1, W3, W4, W5)
    part = _edge_kernel(drug2, tail2, rel2, heads, tails, rels)
    out = _finish(part, drugEmb, W2, b2, gamma, beta)
    return (out, arguments)


# speedup vs baseline: 6.4801x; 1.5364x over previous
import functools

import jax
import jax.numpy as jnp
from jax import lax
from jax.experimental import pallas as pl
from jax.experimental.pallas import tpu as pltpu
from jax.experimental.pallas import tpu_sc as plsc

N = 10000
NP = 10240
E = 320000
D = 128
W = 144
NC = 2
NS = 16
EPW = E // (NC * NS)
C = 80
NCHUNK = EPW // C
RPT = NP // NS


def _prep_body(drugEmb, tailEmb, relEmb, W1, b1, W3, W4, W5,
               drug2, tail2, rel2):
    w1s = jnp.sum(W1[...], axis=1)
    rel2[...] = relEmb[...] * w1s[None, :]
    w5 = W5[...]
    w35 = W3[...] @ w5[:D]
    w45 = W4[...] @ w5[D:]
    p = drugEmb[...] @ w35
    q = tailEmb[...] @ w45
    b1s = jnp.sum(b1[...])
    col = lax.broadcasted_iota(jnp.int32, (N, W - D), 1)
    dext = jnp.where(col == 0, p, 0.0) + jnp.where(col == 1, b1s, 0.0)
    text = jnp.where(col == 0, q, 0.0)
    drug2[:, :D] = drugEmb[...]
    drug2[:, D:] = dext
    tail2[:, :D] = tailEmb[...]
    tail2[:, D:] = text


def _prep(drugEmb, tailEmb, relEmb, W1, b1, W3, W4, W5):
    return pl.pallas_call(
        _prep_body,
        out_shape=(
            jax.ShapeDtypeStruct((N, W), jnp.float32),
            jax.ShapeDtypeStruct((N, W), jnp.float32),
            jax.ShapeDtypeStruct((N, D), jnp.float32),
        ),
    )(drugEmb, tailEmb, relEmb, W1, b1, W3, W4, W5)


SB = 5
NSB = NCHUNK // SB
NSLOT = 3


def _edge_body(drug2, tail2, rel2, idx4, out,
               acc, dbuf, tbuf, rbuf, ibuf, wbuf,
               dsem, tsem, rsem, ssem, isem):
    c = lax.axis_index("c")
    s = lax.axis_index("s")
    w = c * NS + s

    lanes = lax.iota(jnp.int32, 16)
    z16 = jnp.zeros((16,), jnp.float32)
    c128 = jnp.full((16,), D, jnp.int32)
    c129 = jnp.full((16,), D + 1, jnp.int32)

    def _zrow(i, carry):
        def _zcol(k, carry2):
            plsc.store_scatter(tbuf, [i * 16 + lanes, jnp.broadcast_to(k, (16,))], z16)
            return carry2
        return lax.fori_loop(0, W, _zcol, carry)
    lax.fori_loop(0, C // 16, _zrow, 0)

    base = s * RPT
    for j in range(RPT // C):
        pltpu.sync_copy(tbuf.at[pl.ds(0, C)], acc.at[pl.ds(base + j * C, C)])
    plsc.subcore_barrier()

    def _issue_gathers(ci):
        slot = (ci // SB) % NSLOT
        j = ci % SB
        pltpu.async_copy(drug2.at[ibuf.at[slot, j, 0]], dbuf, dsem)
        pltpu.async_copy(rel2.at[ibuf.at[slot, j, 2]], rbuf, rsem)

    def _issue_tail(ci):
        slot = (ci // SB) % NSLOT
        j = ci % SB
        pltpu.async_copy(tail2.at[ibuf.at[slot, j, 1]], tbuf, tsem)

    pltpu.sync_copy(idx4.at[w, 0], ibuf.at[0])
    pltpu.async_copy(idx4.at[w, 1], ibuf.at[1], isem)
    _issue_gathers(0)
    _issue_tail(0)

    def _chunk(ci, carry):
        slot = (ci // SB) % NSLOT
        j = ci % SB
        pltpu.make_async_copy(drug2.at[ibuf.at[slot, j, 0]], dbuf, dsem).wait()
        pltpu.make_async_copy(rel2.at[ibuf.at[slot, j, 2]], rbuf, rsem).wait()
        zf = jnp.zeros((16,), jnp.float32)
        for g in range(C // 16):
            rows = g * 16 + lanes

            def _dot(d, carry):
                a0, a1, a2, a3, d0 = carry
                d1 = d0 + 32
                d2 = d0 + 64
                d3 = d0 + 96
                a0 = a0 + plsc.load_gather(dbuf, [rows, d0]) * plsc.load_gather(rbuf, [rows, d0])
                a1 = a1 + plsc.load_gather(dbuf, [rows, d1]) * plsc.load_gather(rbuf, [rows, d1])
                a2 = a2 + plsc.load_gather(dbuf, [rows, d2]) * plsc.load_gather(rbuf, [rows, d2])
                a3 = a3 + plsc.load_gather(dbuf, [rows, d3]) * plsc.load_gather(rbuf, [rows, d3])
                return (a0, a1, a2, a3, d0 + 1)
            a0, a1, a2, a3, _ = lax.fori_loop(
                0, D // 4, _dot, (zf, zf, zf, zf, jnp.zeros((16,), jnp.int32)),
                unroll=4)
            wbuf[pl.ds(g * 16, 16)] = (a0 + a1) + (a2 + a3)

        pltpu.make_async_copy(tail2.at[ibuf.at[slot, j, 1]], tbuf, tsem).wait()
        for g in range(C // 16):
            rows = g * 16 + lanes
            ph = plsc.load_gather(dbuf, [rows, c128])
            qt = plsc.load_gather(tbuf, [rows, c128])
            x = ph + qt
            att = jnp.maximum(x, 0.01 * x)
            eatt = jnp.exp(att)
            b1v = plsc.load_gather(dbuf, [rows, c129])
            wv = eatt * (wbuf[pl.ds(g * 16, 16)] + b1v)
            wbuf[pl.ds(g * 16, 16)] = wv
            plsc.store_scatter(tbuf, [rows, c128], eatt)

        def _scale(e, carry3):
            wve = plsc.load_gather(wbuf, [jnp.broadcast_to(e, (16,))])
            for k in range(D // 16):
                sl = pl.ds(k * 16, 16)
                tbuf[e, sl] = tbuf[e, sl] * wve
            return carry3
        lax.fori_loop(0, C, _scale, 0, unroll=2)

        pltpu.async_copy(tbuf, acc.at[ibuf.at[slot, j, 0]], ssem, add=False)

        nci = ci + 1
        @pl.when(nci < NCHUNK)
        def _():
            nsb = nci // SB

            @pl.when(nci % SB == 0)
            def _():
                pltpu.make_async_copy(idx4.at[w, nsb], ibuf.at[nsb % NSLOT],
                                      isem).wait()

                @pl.when(nsb + 1 < NSB)
                def _():
                    pltpu.async_copy(idx4.at[w, nsb + 1],
                                     ibuf.at[(nsb + 1) % NSLOT], isem)
            _issue_gathers(nci)

        pltpu.make_async_copy(tbuf, acc.at[ibuf.at[slot, j, 0]], ssem).wait()

        @pl.when(nci < NCHUNK)
        def _():
            _issue_tail(nci)
        return carry
    lax.fori_loop(0, NCHUNK, _chunk, 0)

    plsc.subcore_barrier()
    pltpu.sync_copy(acc.at[pl.ds(base, RPT)], out.at[c, pl.ds(base, RPT)])


_edge_kernel = functools.partial(
    pl.kernel,
    out_type=jax.ShapeDtypeStruct((NC, NP, W), jnp.float32),
    mesh=plsc.VectorSubcoreMesh(core_axis_name="c", subcore_axis_name="s",
                                num_cores=NC, num_subcores=NS),
    compiler_params=pltpu.CompilerParams(use_tc_tiling_on_sc=False,
                                         needs_layout_passes=False),
    scratch_types=[
        pltpu.VMEM_SHARED((NP, W), jnp.float32),
        pltpu.VMEM((C, W), jnp.float32),
        pltpu.VMEM((C, W), jnp.float32),
        pltpu.VMEM((C, D), jnp.float32),
        pltpu.VMEM((NSLOT, SB, 3, C), jnp.int32),
        pltpu.VMEM((C,), jnp.float32),
        pltpu.SemaphoreType.DMA,
        pltpu.SemaphoreType.DMA,
        pltpu.SemaphoreType.DMA,
        pltpu.SemaphoreType.DMA,
        pltpu.SemaphoreType.DMA,
    ],
)(_edge_body)


def _finish_body(part, drugEmb, W2, b2, gamma, beta, out):
    acc = part[0, :N] + part[1, :N]
    denom = acc[:, D:D + 1]
    neigh = acc[:, :D] / jnp.where(denom == 0.0, 1.0, denom)
    lin = drugEmb[...] @ W2[:D] + neigh @ W2[D:] + b2[...][None, :]
    mu = jnp.mean(lin, axis=0, keepdims=True)
    var = jnp.mean((lin - mu) ** 2, axis=0, keepdims=True)
    out[...] = (gamma[...][None, :] * (lin - mu) / jnp.sqrt(var + 1e-5)
                + beta[...][None, :])


def _finish(part, drugEmb, W2, b2, gamma, beta):
    return pl.pallas_call(
        _finish_body,
        out_shape=jax.ShapeDtypeStruct((N, D), jnp.float32),
    )(part, drugEmb, W2, b2, gamma, beta)


def kernel(arguments, DKG, drugEmb, relEmb, tailEmb, W1, b1, W2, b2,
           gamma, beta, W3, W4, W5):
    nw = NC * NS
    idx4 = jnp.stack(
        [DKG[:, 0].reshape(nw, NSB, SB, C),
         DKG[:, 1].reshape(nw, NSB, SB, C),
         DKG[:, 2].reshape(nw, NSB, SB, C)], axis=3)
    drug2, tail2, rel2 = _prep(drugEmb, tailEmb, relEmb, W1, b1, W3, W4, W5)
    part = _edge_kernel(drug2, tail2, rel2, idx4)
    out = _finish(part, drugEmb, W2, b2, gamma, beta)
    return (out, arguments)


# speedup vs baseline: 6.6468x; 1.0257x over previous
import functools

import jax
import jax.numpy as jnp
from jax import lax
from jax.experimental import pallas as pl
from jax.experimental.pallas import tpu as pltpu
from jax.experimental.pallas import tpu_sc as plsc

N = 10000
NP = 10240
E = 320000
D = 128
W = 144
NC = 2
NS = 16
EPW = E // (NC * NS)
C = 80
NCHUNK = EPW // C
RPT = NP // NS


def _prep_body(drugEmb, tailEmb, relEmb, W1, b1, W3, W4, W5,
               drug2, tail2, rel2):
    w1s = jnp.sum(W1[...], axis=1)
    rel2[...] = relEmb[...] * w1s[None, :]
    w5 = W5[...]
    w35 = W3[...] @ w5[:D]
    w45 = W4[...] @ w5[D:]
    p = drugEmb[...] @ w35
    q = tailEmb[...] @ w45
    b1s = jnp.sum(b1[...])
    col = lax.broadcasted_iota(jnp.int32, (N, W - D), 1)
    dext = jnp.where(col == 0, p, 0.0) + jnp.where(col == 1, b1s, 0.0)
    text = jnp.where(col == 0, q, 0.0)
    drug2[:, :D] = drugEmb[...]
    drug2[:, D:] = dext
    tail2[:, :D] = tailEmb[...]
    tail2[:, D:] = text


def _prep(drugEmb, tailEmb, relEmb, W1, b1, W3, W4, W5):
    return pl.pallas_call(
        _prep_body,
        out_shape=(
            jax.ShapeDtypeStruct((N, W), jnp.float32),
            jax.ShapeDtypeStruct((N, W), jnp.float32),
            jax.ShapeDtypeStruct((N, D), jnp.float32),
        ),
    )(drugEmb, tailEmb, relEmb, W1, b1, W3, W4, W5)


SB = 5
NSB = NCHUNK // SB
NSLOT = 3


def _edge_body(drug2, tail2, rel2, idx4, out,
               acc, dbuf, tbuf, rbuf, ibuf, wbuf,
               dsem, tsem, rsem, ssem, isem):
    c = lax.axis_index("c")
    s = lax.axis_index("s")
    w = c * NS + s

    lanes = lax.iota(jnp.int32, 16)
    z16 = jnp.zeros((16,), jnp.float32)
    c128 = jnp.full((16,), D, jnp.int32)
    c129 = jnp.full((16,), D + 1, jnp.int32)

    def _zrow(i, carry):
        def _zcol(k, carry2):
            plsc.store_scatter(tbuf, [i * 16 + lanes, jnp.broadcast_to(k, (16,))], z16)
            return carry2
        return lax.fori_loop(0, W, _zcol, carry)
    lax.fori_loop(0, C // 16, _zrow, 0)

    base = s * RPT
    for j in range(RPT // C):
        pltpu.sync_copy(tbuf.at[pl.ds(0, C)], acc.at[pl.ds(base + j * C, C)])
    plsc.subcore_barrier()

    def _issue_gathers(ci):
        slot = (ci // SB) % NSLOT
        j = ci % SB
        pltpu.async_copy(drug2.at[ibuf.at[slot, j, 0]], dbuf, dsem)
        pltpu.async_copy(rel2.at[ibuf.at[slot, j, 2]], rbuf, rsem)

    def _issue_tail(ci):
        slot = (ci // SB) % NSLOT
        j = ci % SB
        pltpu.async_copy(tail2.at[ibuf.at[slot, j, 1]], tbuf, tsem)

    pltpu.sync_copy(idx4.at[w, 0], ibuf.at[0])
    pltpu.async_copy(idx4.at[w, 1], ibuf.at[1], isem)
    _issue_gathers(0)
    _issue_tail(0)

    def _chunk(ci, carry):
        slot = (ci // SB) % NSLOT
        j = ci % SB
        pltpu.make_async_copy(drug2.at[ibuf.at[slot, j, 0]], dbuf, dsem).wait()
        pltpu.make_async_copy(rel2.at[ibuf.at[slot, j, 2]], rbuf, rsem).wait()
        zf = jnp.zeros((16,), jnp.float32)
        for g in range(C // 16):
            rows = g * 16 + lanes

            def _dot(d, carry):
                a0, a1, a2, a3, d0 = carry
                d1 = d0 + 32
                d2 = d0 + 64
                d3 = d0 + 96
                a0 = a0 + plsc.load_gather(dbuf, [rows, d0]) * plsc.load_gather(rbuf, [rows, d0])
                a1 = a1 + plsc.load_gather(dbuf, [rows, d1]) * plsc.load_gather(rbuf, [rows, d1])
                a2 = a2 + plsc.load_gather(dbuf, [rows, d2]) * plsc.load_gather(rbuf, [rows, d2])
                a3 = a3 + plsc.load_gather(dbuf, [rows, d3]) * plsc.load_gather(rbuf, [rows, d3])
                return (a0, a1, a2, a3, d0 + 1)
            a0, a1, a2, a3, _ = plsc.parallel_loop(
                0, D // 4, unroll=4,
                carry=(zf, zf, zf, zf, jnp.zeros((16,), jnp.int32)))(_dot)
            wbuf[pl.ds(g * 16, 16)] = (a0 + a1) + (a2 + a3)

        pltpu.make_async_copy(tail2.at[ibuf.at[slot, j, 1]], tbuf, tsem).wait()
        for g in range(C // 16):
            rows = g * 16 + lanes
            ph = plsc.load_gather(dbuf, [rows, c128])
            qt = plsc.load_gather(tbuf, [rows, c128])
            x = ph + qt
            att = jnp.maximum(x, 0.01 * x)
            eatt = jnp.exp(att)
            b1v = plsc.load_gather(dbuf, [rows, c129])
            wv = eatt * (wbuf[pl.ds(g * 16, 16)] + b1v)
            wbuf[pl.ds(g * 16, 16)] = wv
            plsc.store_scatter(tbuf, [rows, c128], eatt)

        def _scale(e):
            wve = plsc.load_gather(wbuf, [jnp.broadcast_to(e, (16,))])
            for k in range(D // 16):
                sl = pl.ds(k * 16, 16)
                tbuf[e, sl] = tbuf[e, sl] * wve
        plsc.parallel_loop(0, C, unroll=2)(_scale)

        pltpu.async_copy(tbuf, acc.at[ibuf.at[slot, j, 0]], ssem, add=True)

        nci = ci + 1
        @pl.when(nci < NCHUNK)
        def _():
            nsb = nci // SB

            @pl.when(nci % SB == 0)
            def _():
                pltpu.make_async_copy(idx4.at[w, nsb], ibuf.at[nsb % NSLOT],
                                      isem).wait()

                @pl.when(nsb + 1 < NSB)
                def _():
                    pltpu.async_copy(idx4.at[w, nsb + 1],
                                     ibuf.at[(nsb + 1) % NSLOT], isem)
            _issue_gathers(nci)

        pltpu.make_async_copy(tbuf, acc.at[ibuf.at[slot, j, 0]], ssem).wait()

        @pl.when(nci < NCHUNK)
        def _():
            _issue_tail(nci)
        return carry
    lax.fori_loop(0, NCHUNK, _chunk, 0)

    plsc.subcore_barrier()
    pltpu.sync_copy(acc.at[pl.ds(base, RPT)], out.at[c, pl.ds(base, RPT)])


_edge_kernel = functools.partial(
    pl.kernel,
    out_type=jax.ShapeDtypeStruct((NC, NP, W), jnp.float32),
    mesh=plsc.VectorSubcoreMesh(core_axis_name="c", subcore_axis_name="s",
                                num_cores=NC, num_subcores=NS),
    compiler_params=pltpu.CompilerParams(use_tc_tiling_on_sc=False,
                                         needs_layout_passes=False),
    scratch_types=[
        pltpu.VMEM_SHARED((NP, W), jnp.float32),
        pltpu.VMEM((C, W), jnp.float32),
        pltpu.VMEM((C, W), jnp.float32),
        pltpu.VMEM((C, D), jnp.float32),
        pltpu.VMEM((NSLOT, SB, 3, C), jnp.int32),
        pltpu.VMEM((C,), jnp.float32),
        pltpu.SemaphoreType.DMA,
        pltpu.SemaphoreType.DMA,
        pltpu.SemaphoreType.DMA,
        pltpu.SemaphoreType.DMA,
        pltpu.SemaphoreType.DMA,
    ],
)(_edge_body)


def _finish_body(part, drugEmb, W2, b2, gamma, beta, out):
    acc = part[0, :N] + part[1, :N]
    denom = acc[:, D:D + 1]
    neigh = acc[:, :D] / jnp.where(denom == 0.0, 1.0, denom)
    lin = drugEmb[...] @ W2[:D] + neigh @ W2[D:] + b2[...][None, :]
    mu = jnp.mean(lin, axis=0, keepdims=True)
    var = jnp.mean((lin - mu) ** 2, axis=0, keepdims=True)
    out[...] = (gamma[...][None, :] * (lin - mu) / jnp.sqrt(var + 1e-5)
                + beta[...][None, :])


def _finish(part, drugEmb, W2, b2, gamma, beta):
    return pl.pallas_call(
        _finish_body,
        out_shape=jax.ShapeDtypeStruct((N, D), jnp.float32),
    )(part, drugEmb, W2, b2, gamma, beta)


def kernel(arguments, DKG, drugEmb, relEmb, tailEmb, W1, b1, W2, b2,
           gamma, beta, W3, W4, W5):
    nw = NC * NS
    idx4 = jnp.stack(
        [DKG[:, 0].reshape(nw, NSB, SB, C),
         DKG[:, 1].reshape(nw, NSB, SB, C),
         DKG[:, 2].reshape(nw, NSB, SB, C)], axis=3)
    drug2, tail2, rel2 = _prep(drugEmb, tailEmb, relEmb, W1, b1, W3, W4, W5)
    part = _edge_kernel(drug2, tail2, rel2, idx4)
    out = _finish(part, drugEmb, W2, b2, gamma, beta)
    return (out, arguments)
